# Initial kernel scaffold; baseline (speedup 1.0000x reference)
import jax
import jax.numpy as jnp
from jax.experimental import pallas as pl


def kernel(x, edge_index, W1, b1, W2, b2, W3, b3):
    raise NotImplementedError("write your pallas kernel here")


# speedup vs baseline: 13.7912x; 13.7912x over previous
import functools

import jax
import jax.numpy as jnp
from jax import lax
from jax.experimental import pallas as pl
from jax.experimental.pallas import tpu as pltpu
from jax.experimental.pallas import tpu_sc as plsc

N = 10000
D = 128
E = 320000
NC = 2
NS = 16
NW = NC * NS
CH = 128
NCHUNK = 79
EPW = CH * NCHUNK
EPAD = EPW * NW
NPAD = 10240
RPT = NPAD // NS
ZB = RPT // CH


def _make_mesh():
    return plsc.VectorSubcoreMesh(
        core_axis_name="c", subcore_axis_name="s", num_cores=NC, num_subcores=NS
    )


@functools.cache
def _build_sc_deg():
    @functools.partial(
        pl.kernel,
        mesh=_make_mesh(),
        out_type=jax.ShapeDtypeStruct((NC, NPAD), jnp.float32),
        scratch_types=[
            pltpu.VMEM((CH,), jnp.int32),
            pltpu.VMEM((CH,), jnp.float32),
            pltpu.VMEM((RPT,), jnp.float32),
            pltpu.VMEM_SHARED((NPAD,), jnp.float32),
        ],
    )
    def _sc_deg(dst_hbm, out_hbm, dst_v, ones_v, zeros_v, acc_sh):
        cid = lax.axis_index("c")
        sid = lax.axis_index("s")
        wid = sid * NC + cid

        one16 = jnp.ones((16,), jnp.float32)
        zero16 = jnp.zeros((16,), jnp.float32)
        for j in range(CH // 16):
            ones_v[pl.ds(j * 16, 16)] = one16

        def _zero(i, carry):
            zeros_v[pl.ds(i * 16, 16)] = zero16
            return carry

        lax.fori_loop(0, RPT // 16, _zero, 0)

        row0 = sid * RPT
        pltpu.sync_copy(zeros_v, acc_sh.at[pl.ds(row0, RPT)])
        plsc.subcore_barrier()

        def _chunk(g, carry):
            base = pl.multiple_of(wid * EPW + g * CH, CH)
            pltpu.sync_copy(dst_hbm.at[pl.ds(base, CH)], dst_v)
            pltpu.sync_copy(ones_v, acc_sh.at[dst_v], add=True)
            return carry

        lax.fori_loop(0, NCHUNK, _chunk, 0)

        plsc.subcore_barrier()
        pltpu.sync_copy(acc_sh.at[pl.ds(row0, RPT)], out_hbm.at[cid, pl.ds(row0, RPT)])

    return _sc_deg


@functools.cache
def _build_sc_agg():
    @functools.partial(
        pl.kernel,
        mesh=_make_mesh(),
        out_type=jax.ShapeDtypeStruct((NC, NPAD, D), jnp.float32),
        scratch_types=[
            pltpu.VMEM((CH,), jnp.int32),
            pltpu.VMEM((CH,), jnp.int32),
            pltpu.VMEM((CH, D), jnp.float32),
            pltpu.VMEM_SHARED((NPAD, D), jnp.float32),
            pltpu.SemaphoreType.DMA,
        ],
    )
    def _sc_agg(src_hbm, dst_hbm, hn_hbm, out_hbm, src_v, dst_v, rows_v, acc_sh, sem):
        cid = lax.axis_index("c")
        sid = lax.axis_index("s")
        wid = sid * NC + cid

        zero16 = jnp.zeros((16,), jnp.float32)

        def _zrow(i, carry):
            for j in range(D // 16):
                rows_v[i, pl.ds(j * 16, 16)] = zero16
            return carry

        lax.fori_loop(0, CH, _zrow, 0)

        row0 = sid * RPT
        for k in range(ZB):
            pltpu.sync_copy(rows_v, acc_sh.at[pl.ds(row0 + k * CH, CH)])
        plsc.subcore_barrier()

        def _chunk(g, carry):
            base = pl.multiple_of(wid * EPW + g * CH, CH)
            pltpu.sync_copy(src_hbm.at[pl.ds(base, CH)], src_v)
            pltpu.sync_copy(dst_hbm.at[pl.ds(base, CH)], dst_v)
            pltpu.async_copy(hn_hbm.at[src_v], rows_v, sem).wait()
            pltpu.sync_copy(rows_v, acc_sh.at[dst_v], add=True)
            return carry

        lax.fori_loop(0, NCHUNK, _chunk, 0)

        plsc.subcore_barrier()
        pltpu.sync_copy(acc_sh.at[pl.ds(row0, RPT)], out_hbm.at[cid, pl.ds(row0, RPT)])

    return _sc_agg


BR = 1000
G = N // BR

_row_spec = pl.BlockSpec((BR, D), lambda i: (i, 0))
_col_spec = pl.BlockSpec((BR, 1), lambda i: (i, 0))
_w_spec = pl.BlockSpec((D, D), lambda i: (0, 0))
_b_spec = pl.BlockSpec((1, D), lambda i: (0, 0))


def _t_first_body(d0_ref, d1_ref, x_ref, w_ref, hn_ref, dis_ref):
    deg = d0_ref[...] + d1_ref[...] + 1.0
    dis = lax.rsqrt(deg)
    h = jnp.dot(x_ref[...], w_ref[...], preferred_element_type=jnp.float32)
    dis_ref[...] = dis
    hn_ref[...] = dis * h


_t_first = pl.pallas_call(
    _t_first_body,
    grid=(G,),
    in_specs=[_col_spec, _col_spec, _row_spec, _w_spec],
    out_specs=[_row_spec, _col_spec],
    out_shape=[
        jax.ShapeDtypeStruct((N, D), jnp.float32),
        jax.ShapeDtypeStruct((N, 1), jnp.float32),
    ],
)


def _t_mid_body(p0_ref, p1_ref, hn_ref, dis_ref, b_ref, w_ref, out_ref):
    dis = dis_ref[...]
    t = (p0_ref[...] + p1_ref[...] + hn_ref[...]) * dis + b_ref[...]
    a = jnp.maximum(t, 0.0)
    out_ref[...] = dis * jnp.dot(a, w_ref[...], preferred_element_type=jnp.float32)


_t_mid = pl.pallas_call(
    _t_mid_body,
    grid=(G,),
    in_specs=[_row_spec, _row_spec, _row_spec, _col_spec, _b_spec, _w_spec],
    out_specs=_row_spec,
    out_shape=jax.ShapeDtypeStruct((N, D), jnp.float32),
)


def _t_final_body(p0_ref, p1_ref, hn_ref, dis_ref, b_ref, out_ref):
    t = (p0_ref[...] + p1_ref[...] + hn_ref[...]) * dis_ref[...] + b_ref[...]
    m = jnp.max(t, axis=1, keepdims=True)
    s = jnp.sum(jnp.exp(t - m), axis=1, keepdims=True)
    out_ref[...] = t - m - jnp.log(s)


_t_final = pl.pallas_call(
    _t_final_body,
    grid=(G,),
    in_specs=[_row_spec, _row_spec, _row_spec, _col_spec, _b_spec],
    out_specs=_row_spec,
    out_shape=jax.ShapeDtypeStruct((N, D), jnp.float32),
)


def kernel(x, edge_index, W1, b1, W2, b2, W3, b3):
    sc_deg = _build_sc_deg()
    sc_agg = _build_sc_agg()

    ei = edge_index.astype(jnp.int32)
    pad = EPAD - E
    pr = jnp.arange(pad, dtype=jnp.int32)
    src_p = jnp.concatenate([ei[0], pr % N])
    dst_p = jnp.concatenate([ei[1], N + pr % (NPAD - N)])

    degp = sc_deg(dst_p)
    d0 = degp[0, :N].reshape(N, 1)
    d1 = degp[1, :N].reshape(N, 1)

    hn1, dis = _t_first(d0, d1, x, W1)
    p = sc_agg(src_p, dst_p, hn1)
    hn2 = _t_mid(p[0, :N], p[1, :N], hn1, dis, b1.reshape(1, D), W2)
    p = sc_agg(src_p, dst_p, hn2)
    hn3 = _t_mid(p[0, :N], p[1, :N], hn2, dis, b2.reshape(1, D), W3)
    p = sc_agg(src_p, dst_p, hn3)
    return _t_final(p[0, :N], p[1, :N], hn3, dis, b3.reshape(1, D))


# speedup vs baseline: 27.8856x; 2.0220x over previous
import functools

import jax
import jax.numpy as jnp
from jax import lax
from jax.experimental import pallas as pl
from jax.experimental.pallas import tpu as pltpu
from jax.experimental.pallas import tpu_sc as plsc

N = 10000
D = 128
E = 320000
NC = 2
NS = 16
NW = NC * NS
CH = 128
NB = 2
NCHUNK = 80
EPW = CH * NCHUNK
EPAD = EPW * NW
NPAD = 10240
RPT = NPAD // NS
ZB = RPT // CH


def _make_mesh():
    return plsc.VectorSubcoreMesh(
        core_axis_name="c", subcore_axis_name="s", num_cores=NC, num_subcores=NS
    )


@functools.cache
def _build_sc_deg():
    @functools.partial(
        pl.kernel,
        mesh=_make_mesh(),
        out_type=jax.ShapeDtypeStruct((NC, NPAD), jnp.float32),
        scratch_types=[
            pltpu.VMEM((NCHUNK, CH), jnp.int32),
            pltpu.VMEM((CH,), jnp.float32),
            pltpu.VMEM((RPT,), jnp.float32),
            pltpu.VMEM_SHARED((NPAD,), jnp.float32),
        ],
    )
    def _sc_deg(dst_hbm, out_hbm, dst_st, ones_v, zeros_v, acc_sh):
        cid = lax.axis_index("c")
        sid = lax.axis_index("s")
        wid = sid * NC + cid

        pltpu.sync_copy(dst_hbm.at[wid], dst_st)

        one16 = jnp.ones((16,), jnp.float32)
        zero16 = jnp.zeros((16,), jnp.float32)
        for j in range(CH // 16):
            ones_v[pl.ds(j * 16, 16)] = one16

        def _zero(i, carry):
            zeros_v[pl.ds(i * 16, 16)] = zero16
            return carry

        lax.fori_loop(0, RPT // 16, _zero, 0)

        row0 = sid * RPT
        pltpu.sync_copy(zeros_v, acc_sh.at[pl.ds(row0, RPT)])
        plsc.subcore_barrier()

        def _chunk(g, carry):
            pltpu.sync_copy(ones_v, acc_sh.at[dst_st.at[g]], add=True)
            return carry

        lax.fori_loop(0, NCHUNK, _chunk, 0)

        plsc.subcore_barrier()
        pltpu.sync_copy(acc_sh.at[pl.ds(row0, RPT)], out_hbm.at[cid, pl.ds(row0, RPT)])

    return _sc_deg


@functools.cache
def _build_sc_agg():
    @functools.partial(
        pl.kernel,
        mesh=_make_mesh(),
        out_type=jax.ShapeDtypeStruct((NC, NPAD, D), jnp.float32),
        scratch_types=[
            pltpu.VMEM((NCHUNK, CH), jnp.int32),
            pltpu.VMEM((NB, CH), jnp.int32),
            pltpu.VMEM((NB, CH, D), jnp.float32),
            pltpu.VMEM_SHARED((NPAD, D), jnp.float32),
        ]
        + [pltpu.SemaphoreType.DMA] * (2 * NB),
    )
    def _sc_agg(src_hbm, dst_hbm, hn_hbm, out_hbm, src_st, dst_v, rows_v, acc_sh, *sems):
        gsem, dsem = sems[:NB], sems[NB:]
        cid = lax.axis_index("c")
        sid = lax.axis_index("s")
        wid = sid * NC + cid

        pltpu.sync_copy(src_hbm.at[wid], src_st)

        zero16 = jnp.zeros((16,), jnp.float32)

        def _zrow(i, carry):
            for j in range(D // 16):
                rows_v[0, i, pl.ds(j * 16, 16)] = zero16
            return carry

        lax.fori_loop(0, CH, _zrow, 0)

        row0 = sid * RPT
        for k in range(ZB):
            pltpu.sync_copy(rows_v.at[0], acc_sh.at[pl.ds(row0 + k * CH, CH)])
        plsc.subcore_barrier()

        for b in range(NB):
            pltpu.make_async_copy(dst_hbm.at[wid, b], dst_v.at[b], dsem[b]).start()
            pltpu.make_async_copy(hn_hbm.at[src_st.at[b]], rows_v.at[b], gsem[b]).start()

        def _outer(o, carry):
            for b in range(NB):
                g = o * NB + b
                pltpu.make_async_copy(
                    hn_hbm.at[src_st.at[g]], rows_v.at[b], gsem[b]
                ).wait()
                pltpu.make_async_copy(dst_hbm.at[wid, g], dst_v.at[b], dsem[b]).wait()
                pltpu.sync_copy(rows_v.at[b], acc_sh.at[dst_v.at[b]], add=True)
                gn = g + NB

                @pl.when(gn < NCHUNK)
                def _fire():
                    pltpu.make_async_copy(
                        dst_hbm.at[wid, gn], dst_v.at[b], dsem[b]
                    ).start()
                    pltpu.make_async_copy(
                        hn_hbm.at[src_st.at[gn]], rows_v.at[b], gsem[b]
                    ).start()

            return carry

        lax.fori_loop(0, NCHUNK // NB, _outer, 0)

        plsc.subcore_barrier()
        pltpu.sync_copy(acc_sh.at[pl.ds(row0, RPT)], out_hbm.at[cid, pl.ds(row0, RPT)])

    return _sc_agg


BR = 1000
G = N // BR

_row_spec = pl.BlockSpec((BR, D), lambda i: (i, 0))
_col_spec = pl.BlockSpec((BR, 1), lambda i: (i, 0))
_w_spec = pl.BlockSpec((D, D), lambda i: (0, 0))
_b_spec = pl.BlockSpec((1, D), lambda i: (0, 0))


def _t_first_body(d0_ref, d1_ref, x_ref, w_ref, hn_ref, dis_ref):
    deg = d0_ref[...] + d1_ref[...] + 1.0
    dis = lax.rsqrt(deg)
    h = jnp.dot(x_ref[...], w_ref[...], preferred_element_type=jnp.float32)
    dis_ref[...] = dis
    hn_ref[...] = dis * h


_t_first = pl.pallas_call(
    _t_first_body,
    grid=(G,),
    in_specs=[_col_spec, _col_spec, _row_spec, _w_spec],
    out_specs=[_row_spec, _col_spec],
    out_shape=[
        jax.ShapeDtypeStruct((N, D), jnp.float32),
        jax.ShapeDtypeStruct((N, 1), jnp.float32),
    ],
)


def _t_mid_body(p0_ref, p1_ref, hn_ref, dis_ref, b_ref, w_ref, out_ref):
    dis = dis_ref[...]
    t = (p0_ref[...] + p1_ref[...] + hn_ref[...]) * dis + b_ref[...]
    a = jnp.maximum(t, 0.0)
    out_ref[...] = dis * jnp.dot(a, w_ref[...], preferred_element_type=jnp.float32)


_t_mid = pl.pallas_call(
    _t_mid_body,
    grid=(G,),
    in_specs=[_row_spec, _row_spec, _row_spec, _col_spec, _b_spec, _w_spec],
    out_specs=_row_spec,
    out_shape=jax.ShapeDtypeStruct((N, D), jnp.float32),
)


def _t_final_body(p0_ref, p1_ref, hn_ref, dis_ref, b_ref, out_ref):
    t = (p0_ref[...] + p1_ref[...] + hn_ref[...]) * dis_ref[...] + b_ref[...]
    m = jnp.max(t, axis=1, keepdims=True)
    s = jnp.sum(jnp.exp(t - m), axis=1, keepdims=True)
    out_ref[...] = t - m - jnp.log(s)


_t_final = pl.pallas_call(
    _t_final_body,
    grid=(G,),
    in_specs=[_row_spec, _row_spec, _row_spec, _col_spec, _b_spec],
    out_specs=_row_spec,
    out_shape=jax.ShapeDtypeStruct((N, D), jnp.float32),
)


def kernel(x, edge_index, W1, b1, W2, b2, W3, b3):
    sc_deg = _build_sc_deg()
    sc_agg = _build_sc_agg()

    ei = edge_index.astype(jnp.int32)
    pad = EPAD - E
    pr = jnp.arange(pad, dtype=jnp.int32)
    src_p = jnp.concatenate([ei[0], pr % N]).reshape(NW, NCHUNK, CH)
    dst_p = jnp.concatenate([ei[1], N + pr % (NPAD - N)]).reshape(NW, NCHUNK, CH)

    degp = sc_deg(dst_p)
    d0 = degp[0, :N].reshape(N, 1)
    d1 = degp[1, :N].reshape(N, 1)

    hn1, dis = _t_first(d0, d1, x, W1)
    p = sc_agg(src_p, dst_p, hn1)
    hn2 = _t_mid(p[0, :N], p[1, :N], hn1, dis, b1.reshape(1, D), W2)
    p = sc_agg(src_p, dst_p, hn2)
    hn3 = _t_mid(p[0, :N], p[1, :N], hn2, dis, b2.reshape(1, D), W3)
    p = sc_agg(src_p, dst_p, hn3)
    return _t_final(p[0, :N], p[1, :N], hn3, dis, b3.reshape(1, D))


# speedup vs baseline: 29.9254x; 1.0731x over previous
import functools

import jax
import jax.numpy as jnp
from jax import lax
from jax.experimental import pallas as pl
from jax.experimental.pallas import tpu as pltpu
from jax.experimental.pallas import tpu_sc as plsc

N = 10000
D = 128
E = 320000
NC = 2
NS = 16
NW = NC * NS
EPW = E // NW
CHD = 128
NCHD = EPW // CHD
CH = 104
NCHUNK = 96
TAIL = 16
RB = 3
IB = 6
NA = 10112
RPT_A = NA // NS
ND = 10240
RPT_D = ND // NS


def _make_mesh():
    return plsc.VectorSubcoreMesh(
        core_axis_name="c", subcore_axis_name="s", num_cores=NC, num_subcores=NS
    )


@functools.cache
def _build_sc_deg():
    @functools.partial(
        pl.kernel,
        mesh=_make_mesh(),
        out_type=jax.ShapeDtypeStruct((NC, ND), jnp.float32),
        scratch_types=[
            pltpu.VMEM((2, CHD), jnp.int32),
            pltpu.VMEM((TAIL,), jnp.int32),
            pltpu.VMEM((CHD,), jnp.float32),
            pltpu.VMEM((RPT_D,), jnp.float32),
            pltpu.VMEM_SHARED((ND,), jnp.float32),
        ]
        + [pltpu.SemaphoreType.DMA] * 2,
    )
    def _sc_deg(dst_hbm, out_hbm, dst_st, dst_t, ones_v, zeros_v, acc_sh, *isem):
        cid = lax.axis_index("c")
        sid = lax.axis_index("s")
        wid = sid * NC + cid
        ebase = wid * EPW

        for u in range(2):
            pltpu.make_async_copy(
                dst_hbm.at[pl.ds(pl.multiple_of(ebase + u * CHD, 8), CHD)],
                dst_st.at[u],
                isem[u],
            ).start()

        one16 = jnp.ones((16,), jnp.float32)
        zero16 = jnp.zeros((16,), jnp.float32)
        for j in range(CHD // 16):
            ones_v[pl.ds(j * 16, 16)] = one16

        def _zero(i, carry):
            zeros_v[pl.ds(i * 16, 16)] = zero16
            return carry

        lax.fori_loop(0, RPT_D // 16, _zero, 0)

        row0 = sid * RPT_D
        pltpu.sync_copy(zeros_v, acc_sh.at[pl.ds(row0, RPT_D)])
        plsc.subcore_barrier()

        def _outer(o, carry):
            for u in range(2):
                g = o * 2 + u
                pltpu.make_async_copy(
                    dst_hbm.at[pl.ds(pl.multiple_of(ebase + g * CHD, 8), CHD)],
                    dst_st.at[u],
                    isem[u],
                ).wait()
                pltpu.sync_copy(ones_v, acc_sh.at[dst_st.at[u]], add=True)
                gn = g + 2

                @pl.when(gn < NCHD)
                def _fire():
                    pltpu.make_async_copy(
                        dst_hbm.at[pl.ds(pl.multiple_of(ebase + gn * CHD, 8), CHD)],
                        dst_st.at[u],
                        isem[u],
                    ).start()

            return carry

        lax.fori_loop(0, NCHD // 2, _outer, 0)

        pltpu.sync_copy(dst_hbm.at[pl.ds(pl.multiple_of(ebase + NCHD * CHD, TAIL), TAIL)], dst_t)
        pltpu.sync_copy(ones_v.at[pl.ds(0, TAIL)], acc_sh.at[dst_t], add=True)

        plsc.subcore_barrier()
        pltpu.sync_copy(acc_sh.at[pl.ds(row0, RPT_D)], out_hbm.at[cid, pl.ds(row0, RPT_D)])

    return _sc_deg


@functools.cache
def _build_sc_agg():
    @functools.partial(
        pl.kernel,
        mesh=_make_mesh(),
        out_type=jax.ShapeDtypeStruct((NC, NA, D), jnp.float32),
        scratch_types=[
            pltpu.VMEM((IB, CH), jnp.int32),
            pltpu.VMEM((IB, CH), jnp.int32),
            pltpu.VMEM((TAIL,), jnp.int32),
            pltpu.VMEM((TAIL,), jnp.int32),
            pltpu.VMEM((RB, CH, D), jnp.float32),
            pltpu.VMEM_SHARED((NA, D), jnp.float32),
        ]
        + [pltpu.SemaphoreType.DMA] * (2 * RB + IB),
    )
    def _sc_agg(src_hbm, dst_hbm, hn_hbm, out_hbm, src_st, dst_st, src_t, dst_t, rows_v,
                acc_sh, *sems):
        gsem, ssem, isem = sems[:RB], sems[RB : 2 * RB], sems[2 * RB :]
        cid = lax.axis_index("c")
        sid = lax.axis_index("s")
        wid = sid * NC + cid
        ebase = wid * EPW

        def _idx_copies(g, islot):
            off = pl.ds(pl.multiple_of(ebase + g * CH, 8), CH)
            return (
                pltpu.make_async_copy(src_hbm.at[off], src_st.at[islot], isem[islot]),
                pltpu.make_async_copy(dst_hbm.at[off], dst_st.at[islot], isem[islot]),
            )

        def _gather(g, islot, rslot):
            return pltpu.make_async_copy(
                hn_hbm.at[src_st.at[islot]], rows_v.at[rslot], gsem[rslot]
            )

        def _scatter(islot, rslot):
            return pltpu.make_async_copy(
                rows_v.at[rslot], acc_sh.at[dst_st.at[islot]], ssem[rslot]
            )

        for k in range(4):
            for c in _idx_copies(k, k):
                c.start()

        zero16 = jnp.zeros((16,), jnp.float32)

        def _zrow(i, carry):
            for j in range(D // 16):
                rows_v[0, i, pl.ds(j * 16, 16)] = zero16
            return carry

        lax.fori_loop(0, CH, _zrow, 0)

        row0 = sid * RPT_A
        for k in range(RPT_A // CH):
            pltpu.sync_copy(rows_v.at[0], acc_sh.at[pl.ds(row0 + k * CH, CH)])
        rem = RPT_A - (RPT_A // CH) * CH
        pltpu.sync_copy(
            rows_v.at[0, pl.ds(0, rem)],
            acc_sh.at[pl.ds(row0 + (RPT_A // CH) * CH, rem)],
        )
        plsc.subcore_barrier()

        for c in _idx_copies(0, 0):
            c.wait()
        _gather(0, 0, 0).start()

        def _outer(o, carry):
            for u in range(IB):
                g = o * IB + u
                rs = u % RB
                rs_next = (u + 1) % RB
                is_next = (u + 1) % IB
                is_m2 = (u + IB - 2) % IB

                @pl.when(g >= 2)
                def _wait_scatter_gm2():
                    _scatter(is_m2, rs_next).wait()

                gf = g + 4

                @pl.when(gf < NCHUNK)
                def _refill_idx():
                    for c in _idx_copies(gf, is_m2):
                        c.start()

                gg = g + 1

                @pl.when(gg < NCHUNK)
                def _fire_gather():
                    for c in _idx_copies(gg, is_next):
                        c.wait()
                    _gather(gg, is_next, rs_next).start()

                _gather(g, u, rs).wait()
                pltpu.async_copy(
                    rows_v.at[rs], acc_sh.at[dst_st.at[u]], ssem[rs], add=True
                )

            return carry

        lax.fori_loop(0, NCHUNK // IB, _outer, 0)

        _scatter((NCHUNK - 2) % IB, (NCHUNK - 2) % RB).wait()
        _scatter((NCHUNK - 1) % IB, (NCHUNK - 1) % RB).wait()

        toff = pl.ds(pl.multiple_of(ebase + NCHUNK * CH, TAIL), TAIL)
        pltpu.sync_copy(src_hbm.at[toff], src_t)
        pltpu.sync_copy(dst_hbm.at[toff], dst_t)
        pltpu.async_copy(
            hn_hbm.at[src_t], rows_v.at[0, pl.ds(0, TAIL)], gsem[0]
        ).wait()
        pltpu.sync_copy(rows_v.at[0, pl.ds(0, TAIL)], acc_sh.at[dst_t], add=True)

        plsc.subcore_barrier()
        pltpu.sync_copy(acc_sh.at[pl.ds(row0, RPT_A)], out_hbm.at[cid, pl.ds(row0, RPT_A)])

    return _sc_agg


BR = 1000
G = N // BR

_row_spec = pl.BlockSpec((BR, D), lambda i: (i, 0))
_col_spec = pl.BlockSpec((BR, 1), lambda i: (i, 0))
_w_spec = pl.BlockSpec((D, D), lambda i: (0, 0))
_b_spec = pl.BlockSpec((1, D), lambda i: (0, 0))
_p_spec = pl.BlockSpec((NC, BR, D), lambda i: (0, i, 0))


def _t_first_body(d0_ref, d1_ref, x_ref, w_ref, hn_ref, dis_ref):
    deg = d0_ref[...] + d1_ref[...] + 1.0
    dis = lax.rsqrt(deg)
    h = jnp.dot(x_ref[...], w_ref[...], preferred_element_type=jnp.float32)
    dis_ref[...] = dis
    hn_ref[...] = dis * h


_t_first = pl.pallas_call(
    _t_first_body,
    grid=(G,),
    in_specs=[_col_spec, _col_spec, _row_spec, _w_spec],
    out_specs=[_row_spec, _col_spec],
    out_shape=[
        jax.ShapeDtypeStruct((N, D), jnp.float32),
        jax.ShapeDtypeStruct((N, 1), jnp.float32),
    ],
)


def _t_mid_body(p_ref, hn_ref, dis_ref, b_ref, w_ref, out_ref):
    dis = dis_ref[...]
    t = (p_ref[0] + p_ref[1] + hn_ref[...]) * dis + b_ref[...]
    a = jnp.maximum(t, 0.0)
    out_ref[...] = dis * jnp.dot(a, w_ref[...], preferred_element_type=jnp.float32)


_t_mid = pl.pallas_call(
    _t_mid_body,
    grid=(G,),
    in_specs=[_p_spec, _row_spec, _col_spec, _b_spec, _w_spec],
    out_specs=_row_spec,
    out_shape=jax.ShapeDtypeStruct((N, D), jnp.float32),
)


def _t_final_body(p_ref, hn_ref, dis_ref, b_ref, out_ref):
    t = (p_ref[0] + p_ref[1] + hn_ref[...]) * dis_ref[...] + b_ref[...]
    m = jnp.max(t, axis=1, keepdims=True)
    s = jnp.sum(jnp.exp(t - m), axis=1, keepdims=True)
    out_ref[...] = t - m - jnp.log(s)


_t_final = pl.pallas_call(
    _t_final_body,
    grid=(G,),
    in_specs=[_p_spec, _row_spec, _col_spec, _b_spec],
    out_specs=_row_spec,
    out_shape=jax.ShapeDtypeStruct((N, D), jnp.float32),
)


def kernel(x, edge_index, W1, b1, W2, b2, W3, b3):
    sc_deg = _build_sc_deg()
    sc_agg = _build_sc_agg()

    ei = edge_index.astype(jnp.int32)
    src_p, dst_p = ei[0], ei[1]

    degp = sc_deg(dst_p)
    d0 = degp[0, :N].reshape(N, 1)
    d1 = degp[1, :N].reshape(N, 1)

    hn1, dis = _t_first(d0, d1, x, W1)
    p = sc_agg(src_p, dst_p, hn1)
    hn2 = _t_mid(p, hn1, dis, b1.reshape(1, D), W2)
    p = sc_agg(src_p, dst_p, hn2)
    hn3 = _t_mid(p, hn2, dis, b2.reshape(1, D), W3)
    p = sc_agg(src_p, dst_p, hn3)
    return _t_final(p, hn3, dis, b3.reshape(1, D))


# speedup vs baseline: 32.8281x; 1.0970x over previous
import functools

import jax
import jax.numpy as jnp
from jax import lax
from jax.experimental import pallas as pl
from jax.experimental.pallas import tpu as pltpu
from jax.experimental.pallas import tpu_sc as plsc

N = 10000
D = 128
E = 320000
NC = 2
NS = 16
NW = NC * NS
EPW = E // NW
CHD = 128
NCHD = EPW // CHD
CH = 104
NCHUNK = 96
TAIL = 16
RB = 3
IB = 6
NA = 10112
RPT_A = NA // NS
ND = 10240
RPT_D = ND // NS


def _make_mesh():
    return plsc.VectorSubcoreMesh(
        core_axis_name="c", subcore_axis_name="s", num_cores=NC, num_subcores=NS
    )


@functools.cache
def _build_sc_deg():
    @functools.partial(
        pl.kernel,
        mesh=_make_mesh(),
        out_type=[
            jax.ShapeDtypeStruct((NC, ND), jnp.float32),
            jax.ShapeDtypeStruct((E,), jnp.int32),
            jax.ShapeDtypeStruct((E,), jnp.int32),
        ],
        scratch_types=[
            pltpu.VMEM((NCHD, 2, CHD), jnp.int32),
            pltpu.VMEM((CHD,), jnp.float32),
            pltpu.VMEM((RPT_D,), jnp.float32),
            pltpu.VMEM_SHARED((ND,), jnp.float32),
        ]
        + [pltpu.SemaphoreType.DMA] * 3,
    )
    def _sc_deg(ei_hbm, out_hbm, src_hbm, dst_hbm, st, ones_v, zeros_v, acc_sh,
                isem, ssem, osem):
        cid = lax.axis_index("c")
        sid = lax.axis_index("s")
        wid = sid * NC + cid

        def _load(g):
            coff = pl.ds(pl.multiple_of((g * NW + wid) * CHD, CHD), CHD)
            return pltpu.make_async_copy(ei_hbm.at[pl.ds(0, 2), coff], st.at[g], isem)

        def _copyouts(g):
            coff = pl.ds(pl.multiple_of((g * NW + wid) * CHD, CHD), CHD)
            return (
                pltpu.make_async_copy(st.at[g, 0], src_hbm.at[coff], osem),
                pltpu.make_async_copy(st.at[g, 1], dst_hbm.at[coff], osem),
            )

        def _scatter(g):
            return pltpu.make_async_copy(ones_v, acc_sh.at[st.at[g, 1]], ssem)

        def _fire(g, carry):
            _load(g).start()
            return carry

        lax.fori_loop(0, NCHD, _fire, 0)

        one16 = jnp.ones((16,), jnp.float32)
        zero16 = jnp.zeros((16,), jnp.float32)
        for j in range(CHD // 16):
            ones_v[pl.ds(j * 16, 16)] = one16

        def _zero(i, carry):
            zeros_v[pl.ds(i * 16, 16)] = zero16
            return carry

        lax.fori_loop(0, RPT_D // 16, _zero, 0)

        row0 = sid * RPT_D
        pltpu.sync_copy(zeros_v, acc_sh.at[pl.ds(row0, RPT_D)])
        plsc.subcore_barrier()

        def _go(g, carry):
            _load(g).wait()
            _scatter(g).start(add=True)
            for c in _copyouts(g):
                c.start()
            return carry

        lax.fori_loop(0, NCHD, _go, 0)

        def _drain(g, carry):
            _scatter(g).wait()
            for c in _copyouts(g):
                c.wait()
            return carry

        lax.fori_loop(0, NCHD, _drain, 0)

        @pl.when(wid < 4)
        def _extra():
            coff = pl.ds(pl.multiple_of((NCHD * NW + wid) * CHD, CHD), CHD)
            pltpu.sync_copy(ei_hbm.at[pl.ds(0, 2), coff], st.at[0])
            pltpu.sync_copy(ones_v, acc_sh.at[st.at[0, 1]], add=True)
            pltpu.sync_copy(st.at[0, 0], src_hbm.at[coff])
            pltpu.sync_copy(st.at[0, 1], dst_hbm.at[coff])

        plsc.subcore_barrier()
        pltpu.sync_copy(acc_sh.at[pl.ds(row0, RPT_D)], out_hbm.at[cid, pl.ds(row0, RPT_D)])

    return _sc_deg


@functools.cache
def _build_sc_agg():
    @functools.partial(
        pl.kernel,
        mesh=_make_mesh(),
        out_type=jax.ShapeDtypeStruct((NC, NA, D), jnp.float32),
        scratch_types=[
            pltpu.VMEM((IB, CH), jnp.int32),
            pltpu.VMEM((IB, CH), jnp.int32),
            pltpu.VMEM((TAIL,), jnp.int32),
            pltpu.VMEM((TAIL,), jnp.int32),
            pltpu.VMEM((RB, CH, D), jnp.float32),
            pltpu.VMEM_SHARED((NA, D), jnp.float32),
        ]
        + [pltpu.SemaphoreType.DMA] * (2 * RB + IB),
    )
    def _sc_agg(src_hbm, dst_hbm, hn_hbm, out_hbm, src_st, dst_st, src_t, dst_t, rows_v,
                acc_sh, *sems):
        gsem, ssem, isem = sems[:RB], sems[RB : 2 * RB], sems[2 * RB :]
        cid = lax.axis_index("c")
        sid = lax.axis_index("s")
        wid = sid * NC + cid
        ebase = wid * EPW

        def _idx_copies(g, islot):
            off = pl.ds(pl.multiple_of(ebase + g * CH, 8), CH)
            return (
                pltpu.make_async_copy(src_hbm.at[off], src_st.at[islot], isem[islot]),
                pltpu.make_async_copy(dst_hbm.at[off], dst_st.at[islot], isem[islot]),
            )

        def _gather(g, islot, rslot):
            return pltpu.make_async_copy(
                hn_hbm.at[src_st.at[islot]], rows_v.at[rslot], gsem[rslot]
            )

        def _scatter(islot, rslot):
            return pltpu.make_async_copy(
                rows_v.at[rslot], acc_sh.at[dst_st.at[islot]], ssem[rslot]
            )

        for k in range(4):
            for c in _idx_copies(k, k):
                c.start()

        zero16 = jnp.zeros((16,), jnp.float32)

        def _zrow(i, carry):
            for j in range(D // 16):
                rows_v[0, i, pl.ds(j * 16, 16)] = zero16
            return carry

        lax.fori_loop(0, CH, _zrow, 0)

        row0 = sid * RPT_A
        for k in range(RPT_A // CH):
            pltpu.sync_copy(rows_v.at[0], acc_sh.at[pl.ds(row0 + k * CH, CH)])
        rem = RPT_A - (RPT_A // CH) * CH
        pltpu.sync_copy(
            rows_v.at[0, pl.ds(0, rem)],
            acc_sh.at[pl.ds(row0 + (RPT_A // CH) * CH, rem)],
        )
        plsc.subcore_barrier()

        for c in _idx_copies(0, 0):
            c.wait()
        _gather(0, 0, 0).start()

        def _outer(o, carry):
            for u in range(IB):
                g = o * IB + u
                rs = u % RB
                rs_next = (u + 1) % RB
                is_next = (u + 1) % IB
                is_m2 = (u + IB - 2) % IB

                @pl.when(g >= 2)
                def _wait_scatter_gm2():
                    _scatter(is_m2, rs_next).wait()

                gf = g + 4

                @pl.when(gf < NCHUNK)
                def _refill_idx():
                    for c in _idx_copies(gf, is_m2):
                        c.start()

                gg = g + 1

                @pl.when(gg < NCHUNK)
                def _fire_gather():
                    for c in _idx_copies(gg, is_next):
                        c.wait()
                    _gather(gg, is_next, rs_next).start()

                _gather(g, u, rs).wait()
                pltpu.async_copy(
                    rows_v.at[rs], acc_sh.at[dst_st.at[u]], ssem[rs], add=True
                )

            return carry

        lax.fori_loop(0, NCHUNK // IB, _outer, 0)

        _scatter((NCHUNK - 2) % IB, (NCHUNK - 2) % RB).wait()
        _scatter((NCHUNK - 1) % IB, (NCHUNK - 1) % RB).wait()

        toff = pl.ds(pl.multiple_of(ebase + NCHUNK * CH, TAIL), TAIL)
        pltpu.sync_copy(src_hbm.at[toff], src_t)
        pltpu.sync_copy(dst_hbm.at[toff], dst_t)
        pltpu.async_copy(
            hn_hbm.at[src_t], rows_v.at[0, pl.ds(0, TAIL)], gsem[0]
        ).wait()
        pltpu.sync_copy(rows_v.at[0, pl.ds(0, TAIL)], acc_sh.at[dst_t], add=True)

        plsc.subcore_barrier()
        pltpu.sync_copy(acc_sh.at[pl.ds(row0, RPT_A)], out_hbm.at[cid, pl.ds(row0, RPT_A)])

    return _sc_agg


BR = 1000
G = N // BR

_row_spec = pl.BlockSpec((BR, D), lambda i: (i, 0))
_col_spec = pl.BlockSpec((BR, 1), lambda i: (i, 0))
_w_spec = pl.BlockSpec((D, D), lambda i: (0, 0))
_b_spec = pl.BlockSpec((1, D), lambda i: (0, 0))
_p_spec = pl.BlockSpec((NC, BR, D), lambda i: (0, i, 0))


def _t_first_body(d_ref, x_ref, w_ref, hn_ref, dis_ref):
    dis = lax.rsqrt(d_ref[...])
    h = jnp.dot(x_ref[...], w_ref[...], preferred_element_type=jnp.float32)
    dis_ref[...] = dis
    hn_ref[...] = dis * h


_t_first = pl.pallas_call(
    _t_first_body,
    grid=(G,),
    in_specs=[_col_spec, _row_spec, _w_spec],
    out_specs=[_row_spec, _col_spec],
    out_shape=[
        jax.ShapeDtypeStruct((N, D), jnp.float32),
        jax.ShapeDtypeStruct((N, 1), jnp.float32),
    ],
)


def _t_mid_body(p_ref, hn_ref, dis_ref, b_ref, w_ref, out_ref):
    dis = dis_ref[...]
    t = (p_ref[0] + p_ref[1] + hn_ref[...]) * dis + b_ref[...]
    a = jnp.maximum(t, 0.0)
    out_ref[...] = dis * jnp.dot(a, w_ref[...], preferred_element_type=jnp.float32)


_t_mid = pl.pallas_call(
    _t_mid_body,
    grid=(G,),
    in_specs=[_p_spec, _row_spec, _col_spec, _b_spec, _w_spec],
    out_specs=_row_spec,
    out_shape=jax.ShapeDtypeStruct((N, D), jnp.float32),
)


def _t_final_body(p_ref, hn_ref, dis_ref, b_ref, out_ref):
    t = (p_ref[0] + p_ref[1] + hn_ref[...]) * dis_ref[...] + b_ref[...]
    m = jnp.max(t, axis=1, keepdims=True)
    s = jnp.sum(jnp.exp(t - m), axis=1, keepdims=True)
    out_ref[...] = t - m - jnp.log(s)


_t_final = pl.pallas_call(
    _t_final_body,
    grid=(G,),
    in_specs=[_p_spec, _row_spec, _col_spec, _b_spec],
    out_specs=_row_spec,
    out_shape=jax.ShapeDtypeStruct((N, D), jnp.float32),
)


def kernel(x, edge_index, W1, b1, W2, b2, W3, b3):
    sc_deg = _build_sc_deg()
    sc_agg = _build_sc_agg()

    ei = edge_index.astype(jnp.int32)

    degp, src_p, dst_p = sc_deg(ei)
    d = (degp[0, :N] + degp[1, :N] + 1.0).reshape(N, 1)

    hn1, dis = _t_first(d, x, W1)
    p = sc_agg(src_p, dst_p, hn1)
    hn2 = _t_mid(p, hn1, dis, b1.reshape(1, D), W2)
    p = sc_agg(src_p, dst_p, hn2)
    hn3 = _t_mid(p, hn2, dis, b2.reshape(1, D), W3)
    p = sc_agg(src_p, dst_p, hn3)
    return _t_final(p, hn3, dis, b3.reshape(1, D))


# speedup vs baseline: 33.5236x; 1.0212x over previous
import functools

import jax
import jax.numpy as jnp
from jax import lax
from jax.experimental import pallas as pl
from jax.experimental.pallas import tpu as pltpu
from jax.experimental.pallas import tpu_sc as plsc

N = 10000
D = 128
E = 320000
NC = 2
NS = 16
NW = NC * NS
EPW = E // NW
CHD = 128
NCHD = EPW // CHD
CH = 104
NCHUNK = 96
TAIL = 16
RB = 3
IB = 6
NA = 10112
RPT_A = NA // NS
ND = 10240
RPT_D = ND // NS


def _make_mesh():
    return plsc.VectorSubcoreMesh(
        core_axis_name="c", subcore_axis_name="s", num_cores=NC, num_subcores=NS
    )


@functools.cache
def _build_sc_deg():
    @functools.partial(
        pl.kernel,
        mesh=_make_mesh(),
        out_type=[
            jax.ShapeDtypeStruct((NC, ND), jnp.float32),
            jax.ShapeDtypeStruct((E,), jnp.int32),
            jax.ShapeDtypeStruct((E,), jnp.int32),
        ],
        scratch_types=[
            pltpu.VMEM((NCHD, 2, CHD), jnp.int32),
            pltpu.VMEM((CHD,), jnp.float32),
            pltpu.VMEM((RPT_D,), jnp.float32),
            pltpu.VMEM_SHARED((ND,), jnp.float32),
        ]
        + [pltpu.SemaphoreType.DMA] * 3,
    )
    def _sc_deg(ei_hbm, out_hbm, src_hbm, dst_hbm, st, ones_v, zeros_v, acc_sh,
                isem, ssem, osem):
        cid = lax.axis_index("c")
        sid = lax.axis_index("s")
        wid = sid * NC + cid

        def _load(g):
            coff = pl.ds(pl.multiple_of((g * NW + wid) * CHD, CHD), CHD)
            return pltpu.make_async_copy(ei_hbm.at[pl.ds(0, 2), coff], st.at[g], isem)

        def _copyouts(g):
            coff = pl.ds(pl.multiple_of((g * NW + wid) * CHD, CHD), CHD)
            return (
                pltpu.make_async_copy(st.at[g, 0], src_hbm.at[coff], osem),
                pltpu.make_async_copy(st.at[g, 1], dst_hbm.at[coff], osem),
            )

        def _scatter(g):
            return pltpu.make_async_copy(ones_v, acc_sh.at[st.at[g, 1]], ssem)

        def _fire(g, carry):
            _load(g).start()
            return carry

        lax.fori_loop(0, NCHD, _fire, 0)

        one16 = jnp.ones((16,), jnp.float32)
        zero16 = jnp.zeros((16,), jnp.float32)
        for j in range(CHD // 16):
            ones_v[pl.ds(j * 16, 16)] = one16

        def _zero(i, carry):
            zeros_v[pl.ds(i * 16, 16)] = zero16
            return carry

        lax.fori_loop(0, RPT_D // 16, _zero, 0)

        row0 = sid * RPT_D
        pltpu.sync_copy(zeros_v, acc_sh.at[pl.ds(row0, RPT_D)])
        plsc.subcore_barrier()

        def _go(g, carry):
            _load(g).wait()
            _scatter(g).start(add=True)
            for c in _copyouts(g):
                c.start()
            return carry

        lax.fori_loop(0, NCHD, _go, 0)

        def _drain(g, carry):
            _scatter(g).wait()
            for c in _copyouts(g):
                c.wait()
            return carry

        lax.fori_loop(0, NCHD, _drain, 0)

        @pl.when(wid < 4)
        def _extra():
            coff = pl.ds(pl.multiple_of((NCHD * NW + wid) * CHD, CHD), CHD)
            pltpu.sync_copy(ei_hbm.at[pl.ds(0, 2), coff], st.at[0])
            pltpu.sync_copy(ones_v, acc_sh.at[st.at[0, 1]], add=True)
            pltpu.sync_copy(st.at[0, 0], src_hbm.at[coff])
            pltpu.sync_copy(st.at[0, 1], dst_hbm.at[coff])

        plsc.subcore_barrier()
        pltpu.sync_copy(acc_sh.at[pl.ds(row0, RPT_D)], out_hbm.at[cid, pl.ds(row0, RPT_D)])

    return _sc_deg


@functools.cache
def _build_sc_agg():
    @functools.partial(
        pl.kernel,
        mesh=_make_mesh(),
        out_type=jax.ShapeDtypeStruct((NC, NA, D), jnp.float32),
        scratch_types=[
            pltpu.VMEM((IB, CH), jnp.int32),
            pltpu.VMEM((IB, CH), jnp.int32),
            pltpu.VMEM((TAIL,), jnp.int32),
            pltpu.VMEM((TAIL,), jnp.int32),
            pltpu.VMEM((RB, CH, D), jnp.float32),
            pltpu.VMEM_SHARED((NA, D), jnp.float32),
        ]
        + [pltpu.SemaphoreType.DMA] * (2 * RB + IB),
    )
    def _sc_agg(src_hbm, dst_hbm, hn_hbm, out_hbm, src_st, dst_st, src_t, dst_t, rows_v,
                acc_sh, *sems):
        gsem, ssem, isem = sems[:RB], sems[RB : 2 * RB], sems[2 * RB :]
        cid = lax.axis_index("c")
        sid = lax.axis_index("s")
        wid = sid * NC + cid
        ebase = wid * EPW

        def _idx_copies(g, islot):
            off = pl.ds(pl.multiple_of(ebase + g * CH, 8), CH)
            return (
                pltpu.make_async_copy(src_hbm.at[off], src_st.at[islot], isem[islot]),
                pltpu.make_async_copy(dst_hbm.at[off], dst_st.at[islot], isem[islot]),
            )

        def _gather(g, islot, rslot):
            return pltpu.make_async_copy(
                hn_hbm.at[src_st.at[islot]], rows_v.at[rslot], gsem[rslot]
            )

        def _scatter(islot, rslot):
            return pltpu.make_async_copy(
                rows_v.at[rslot], acc_sh.at[dst_st.at[islot]], ssem[rslot]
            )

        for k in range(4):
            for c in _idx_copies(k, k):
                c.start()

        zero16 = jnp.zeros((16,), jnp.float32)

        def _zrow(i, carry):
            for j in range(D // 16):
                rows_v[0, i, pl.ds(j * 16, 16)] = zero16
            return carry

        lax.fori_loop(0, CH, _zrow, 0)

        row0 = sid * RPT_A
        for k in range(RPT_A // CH):
            pltpu.sync_copy(rows_v.at[0], acc_sh.at[pl.ds(row0 + k * CH, CH)])
        rem = RPT_A - (RPT_A // CH) * CH
        pltpu.sync_copy(
            rows_v.at[0, pl.ds(0, rem)],
            acc_sh.at[pl.ds(row0 + (RPT_A // CH) * CH, rem)],
        )
        plsc.subcore_barrier()

        for c in _idx_copies(0, 0):
            c.wait()
        _gather(0, 0, 0).start()

        def _outer(o, carry):
            for u in range(IB):
                g = o * IB + u
                rs = u % RB
                rs_next = (u + 1) % RB
                is_next = (u + 1) % IB
                is_m2 = (u + IB - 2) % IB

                @pl.when(g >= 2)
                def _wait_scatter_gm2():
                    _scatter(is_m2, rs_next).wait()

                gf = g + 4

                @pl.when(gf < NCHUNK)
                def _refill_idx():
                    for c in _idx_copies(gf, is_m2):
                        c.start()

                gg = g + 1

                @pl.when(gg < NCHUNK)
                def _fire_gather():
                    for c in _idx_copies(gg, is_next):
                        c.wait()
                    _gather(gg, is_next, rs_next).start()

                _gather(g, u, rs).wait()
                pltpu.async_copy(
                    rows_v.at[rs], acc_sh.at[dst_st.at[u]], ssem[rs], add=True
                )

            return carry

        lax.fori_loop(0, NCHUNK // IB, _outer, 0)

        _scatter((NCHUNK - 2) % IB, (NCHUNK - 2) % RB).wait()
        _scatter((NCHUNK - 1) % IB, (NCHUNK - 1) % RB).wait()

        toff = pl.ds(pl.multiple_of(ebase + NCHUNK * CH, TAIL), TAIL)
        pltpu.sync_copy(src_hbm.at[toff], src_t)
        pltpu.sync_copy(dst_hbm.at[toff], dst_t)
        pltpu.async_copy(
            hn_hbm.at[src_t], rows_v.at[0, pl.ds(0, TAIL)], gsem[0]
        ).wait()
        pltpu.sync_copy(rows_v.at[0, pl.ds(0, TAIL)], acc_sh.at[dst_t], add=True)

        plsc.subcore_barrier()
        pltpu.sync_copy(acc_sh.at[pl.ds(row0, RPT_A)], out_hbm.at[cid, pl.ds(row0, RPT_A)])

    return _sc_agg


BR = 2000
G = N // BR

_row_spec = pl.BlockSpec((BR, D), lambda i: (i, 0))
_col_spec = pl.BlockSpec((BR, 1), lambda i: (i, 0))
_w_spec = pl.BlockSpec((D, D), lambda i: (0, 0))
_b_spec = pl.BlockSpec((1, D), lambda i: (0, 0))
_p_spec = pl.BlockSpec((NC, BR, D), lambda i: (0, i, 0))


def _t_first_body(d_ref, x_ref, w_ref, hn_ref, dis_ref):
    dis = lax.rsqrt(d_ref[...])
    h = jnp.dot(x_ref[...], w_ref[...], preferred_element_type=jnp.float32)
    dis_ref[...] = dis
    hn_ref[...] = dis * h


_t_first = pl.pallas_call(
    _t_first_body,
    grid=(G,),
    in_specs=[_col_spec, _row_spec, _w_spec],
    out_specs=[_row_spec, _col_spec],
    out_shape=[
        jax.ShapeDtypeStruct((N, D), jnp.float32),
        jax.ShapeDtypeStruct((N, 1), jnp.float32),
    ],
)


def _t_mid_body(p_ref, hn_ref, dis_ref, b_ref, w_ref, out_ref):
    dis = dis_ref[...]
    t = (p_ref[0] + p_ref[1] + hn_ref[...]) * dis + b_ref[...]
    a = jnp.maximum(t, 0.0)
    out_ref[...] = dis * jnp.dot(a, w_ref[...], preferred_element_type=jnp.float32)


_t_mid = pl.pallas_call(
    _t_mid_body,
    grid=(G,),
    in_specs=[_p_spec, _row_spec, _col_spec, _b_spec, _w_spec],
    out_specs=_row_spec,
    out_shape=jax.ShapeDtypeStruct((N, D), jnp.float32),
)


def _t_final_body(p_ref, hn_ref, dis_ref, b_ref, out_ref):
    t = (p_ref[0] + p_ref[1] + hn_ref[...]) * dis_ref[...] + b_ref[...]
    m = jnp.max(t, axis=1, keepdims=True)
    s = jnp.sum(jnp.exp(t - m), axis=1, keepdims=True)
    out_ref[...] = t - m - jnp.log(s)


_t_final = pl.pallas_call(
    _t_final_body,
    grid=(G,),
    in_specs=[_p_spec, _row_spec, _col_spec, _b_spec],
    out_specs=_row_spec,
    out_shape=jax.ShapeDtypeStruct((N, D), jnp.float32),
)


def kernel(x, edge_index, W1, b1, W2, b2, W3, b3):
    sc_deg = _build_sc_deg()
    sc_agg = _build_sc_agg()

    ei = edge_index.astype(jnp.int32)

    degp, src_p, dst_p = sc_deg(ei)
    d = (degp[0, :N] + degp[1, :N] + 1.0).reshape(N, 1)

    hn1, dis = _t_first(d, x, W1)
    p = sc_agg(src_p, dst_p, hn1)
    hn2 = _t_mid(p, hn1, dis, b1.reshape(1, D), W2)
    p = sc_agg(src_p, dst_p, hn2)
    hn3 = _t_mid(p, hn2, dis, b2.reshape(1, D), W3)
    p = sc_agg(src_p, dst_p, hn3)
    return _t_final(p, hn3, dis, b3.reshape(1, D))
